# Initial kernel scaffold; baseline (speedup 1.0000x reference)
import jax
import jax.numpy as jnp
from jax.experimental import pallas as pl


def kernel(x, pos, edge_index, lframes, batch, W1_0, b1_0, W2_0, b2_0, W3_0, b3_0, W1_1, b1_1, W2_1, b2_1, W3_1, b3_1):
    raise NotImplementedError("write your pallas kernel here")


# speedup vs baseline: 1.7999x; 1.7999x over previous
import functools

import jax
import jax.numpy as jnp
from jax import lax
from jax.experimental import pallas as pl
from jax.experimental.pallas import tpu as pltpu
from jax.experimental.pallas import tpu_sc as plsc

N = 10000
D = 256
E = 160000
NUM_RADIAL = 32
CUTOFF = 5.0
TW = D + 128
GF = 48

NC = 2
NS = 16
NW = NC * NS

EPW = E // NW
GCH = 40
GITER = EPW // GCH

NPW = 312
NPW_LAST = N - (NW - 1) * NPW
ACC_ROWS = 336
DUMMY = 332
DCH = 800
DITER = E // DCH
LCAP = 6400
SCH = 80

_f32 = jnp.float32


def _mesh():
  return plsc.VectorSubcoreMesh(
      core_axis_name="c", subcore_axis_name="s", num_cores=NC, num_subcores=NS)


def _node_mm_body(x_ref, wa_ref, wb_ref, b_ref, ng_ref, ta_ref, tb_ref):
  xv = x_ref[...]
  ng = ng_ref[...]
  ta_ref[:, :D] = jnp.dot(xv, wa_ref[...], preferred_element_type=_f32)
  ta_ref[:, D:] = ng
  tb_ref[:, :D] = jnp.dot(xv, wb_ref[...], preferred_element_type=_f32) + b_ref[...]
  tb_ref[:, D:] = ng


@functools.lru_cache(maxsize=None)
def _node_mm():
  TN = 1000
  grid = N // TN
  return pl.pallas_call(
      _node_mm_body,
      grid=(grid,),
      in_specs=[
          pl.BlockSpec((TN, D), lambda i: (i, 0)),
          pl.BlockSpec((D, D), lambda i: (0, 0)),
          pl.BlockSpec((D, D), lambda i: (0, 0)),
          pl.BlockSpec((1, D), lambda i: (0, 0)),
          pl.BlockSpec((TN, 128), lambda i: (i, 0)),
      ],
      out_specs=[
          pl.BlockSpec((TN, TW), lambda i: (i, 0)),
          pl.BlockSpec((TN, TW), lambda i: (i, 0)),
      ],
      out_shape=[
          jax.ShapeDtypeStruct((N, TW), _f32),
          jax.ShapeDtypeStruct((N, TW), _f32),
      ],
  )


def _gather_body(ta, tb, src, dst, ga, gb, sidx, didx, bufa, bufb, sem):
  wid = lax.axis_index("s") * NC + lax.axis_index("c")
  base = wid * EPW

  def chunk(i, carry):
    off = base + i * GCH
    pltpu.sync_copy(src.at[pl.ds(off, GCH)], sidx)
    pltpu.sync_copy(dst.at[pl.ds(off, GCH)], didx)
    ca = pltpu.async_copy(ta.at[sidx], bufa, sem)
    cb = pltpu.async_copy(tb.at[didx], bufb, sem)
    ca.wait()
    cb.wait()
    pltpu.sync_copy(bufa, ga.at[pl.ds(off, GCH)])
    pltpu.sync_copy(bufb, gb.at[pl.ds(off, GCH)])
    return carry

  lax.fori_loop(0, GITER, chunk, 0)


@functools.lru_cache(maxsize=None)
def _gather():
  return pl.kernel(
      _gather_body,
      out_type=[
          jax.ShapeDtypeStruct((E, TW), _f32),
          jax.ShapeDtypeStruct((E, TW), _f32),
      ],
      mesh=_mesh(),
      compiler_params=pltpu.CompilerParams(needs_layout_passes=False),
      scratch_types=[
          pltpu.VMEM((GCH,), jnp.int32),
          pltpu.VMEM((GCH,), jnp.int32),
          pltpu.VMEM((GCH, TW), _f32),
          pltpu.VMEM((GCH, TW), _f32),
          pltpu.SemaphoreType.DMA,
      ],
  )


def _silu(v):
  return v * (1.0 / (1.0 + jnp.exp(-v)))


def _edge_mlp_body(ga_ref, gb_ref, w1_ref, w2_ref, b2_ref, w3_ref, b3_ref,
                   m_ref):
  gsT = ga_ref[:, D:D + 16].T
  gdT = gb_ref[:, D:D + 16].T
  vec = gsT[0:3, :] - gdT[0:3, :]
  L = gdT[3:12, :]
  v0 = vec[0:1, :]
  v1 = vec[1:2, :]
  v2 = vec[2:3, :]
  vl0 = L[0:1, :] * v0 + L[1:2, :] * v1 + L[2:3, :] * v2
  vl1 = L[3:4, :] * v0 + L[4:5, :] * v1 + L[5:6, :] * v2
  vl2 = L[6:7, :] * v0 + L[7:8, :] * v1 + L[8:9, :] * v2
  r2 = vl0 * vl0 + vl1 * vl1 + vl2 * vl2
  r = jnp.sqrt(r2)
  rinv = 1.0 / (r + 1e-8)
  u0 = vl0 * rinv
  u1 = vl1 * rinv
  u2 = vl2 * rinv
  centers = lax.broadcasted_iota(jnp.int32, (NUM_RADIAL, 1), 0).astype(_f32) * (
      CUTOFF / (NUM_RADIAL - 1))
  dr = r - centers
  rad = jnp.exp(-10.0 * dr * dr)
  us = [u0, u1, u2]
  outer = [us[i] * us[j] for i in range(3) for j in range(3)]
  te = r.shape[1]
  geoT = jnp.concatenate(
      [rad, u0, u1, u2] + outer + [jnp.zeros((4, te), _f32)], axis=0)
  term = lax.dot_general(geoT, w1_ref[...], (((0,), (0,)), ((), ())),
                         preferred_element_type=_f32)
  pre1 = ga_ref[:, :D] + gb_ref[:, :D] + term
  h1 = _silu(pre1)
  h2 = _silu(jnp.dot(h1, w2_ref[...], preferred_element_type=_f32) + b2_ref[...])
  m_ref[...] = jnp.dot(h2, w3_ref[...], preferred_element_type=_f32) + b3_ref[...]


@functools.lru_cache(maxsize=None)
def _edge_mlp():
  TE = 2000
  grid = E // TE
  return pl.pallas_call(
      _edge_mlp_body,
      grid=(grid,),
      in_specs=[
          pl.BlockSpec((TE, TW), lambda i: (i, 0)),
          pl.BlockSpec((TE, TW), lambda i: (i, 0)),
          pl.BlockSpec((GF, D), lambda i: (0, 0)),
          pl.BlockSpec((D, D), lambda i: (0, 0)),
          pl.BlockSpec((1, D), lambda i: (0, 0)),
          pl.BlockSpec((D, D), lambda i: (0, 0)),
          pl.BlockSpec((1, D), lambda i: (0, 0)),
      ],
      out_specs=pl.BlockSpec((TE, D), lambda i: (i, 0)),
      out_shape=jax.ShapeDtypeStruct((E, D), _f32),
  )


def _scatter_body(m, dst, out, didx, eids, lids, gbuf, acc, sem):
  w = lax.axis_index("s") * NC + lax.axis_index("c")
  nodebase = w * NPW

  zero16 = jnp.zeros((16,), _f32)

  def zrow(r, carry):
    for jj in range(D // 16):
      acc[r, pl.ds(jj * 16, 16)] = zero16
    return carry

  lax.fori_loop(0, ACC_ROWS, zrow, 0)

  iota16 = lax.iota(jnp.int32, 16)

  bound = jnp.where(w == NW - 1, NPW_LAST, NPW)

  def scan_chunk(i, off):
    cbase = i * DCH
    pltpu.sync_copy(dst.at[pl.ds(cbase, DCH)], didx)

    def group(g, off):
      v = didx[pl.ds(g * 16, 16)] - nodebase
      ok = (v >= 0) & (v < bound)
      okx = ok.astype(jnp.int32)
      csum = plsc.cumsum(okx)
      pos = jnp.where(ok, off + csum - okx, LCAP - 1)
      plsc.store_scatter(eids, [pos], cbase + g * 16 + iota16)
      plsc.store_scatter(lids, [pos], v)
      return off + jnp.sum(okx)

    return lax.fori_loop(0, DCH // 16, group, off)

  cnt = lax.fori_loop(0, DITER, scan_chunk, 0)

  for g in range(SCH // 16):
    eids[pl.ds(cnt + g * 16, 16)] = jnp.zeros((16,), jnp.int32)
    lids[pl.ds(cnt + g * 16, 16)] = jnp.full((16,), DUMMY, jnp.int32)

  nchunk = (cnt + SCH - 1) // SCH

  def acc_chunk(ch, carry):
    pltpu.async_copy(m.at[eids.at[pl.ds(ch * SCH, SCH)]], gbuf, sem).wait()

    def row(k, carry):
      d = lids[pl.ds(ch * SCH + k, 16)][0]
      for jj in range(D // 16):
        plsc.addupdate(acc.at[d, pl.ds(jj * 16, 16)],
                       gbuf[k, pl.ds(jj * 16, 16)])
      return carry

    lax.fori_loop(0, SCH, row, 0)
    return carry

  lax.fori_loop(0, nchunk, acc_chunk, 0)

  @pl.when(w < NW - 1)
  def _():
    pltpu.sync_copy(acc.at[pl.ds(0, NPW)], out.at[pl.ds(nodebase, NPW)])

  @pl.when(w == NW - 1)
  def _():
    pltpu.sync_copy(acc.at[pl.ds(0, NPW_LAST)],
                    out.at[pl.ds(nodebase, NPW_LAST)])


@functools.lru_cache(maxsize=None)
def _scatter():
  return pl.kernel(
      _scatter_body,
      out_type=jax.ShapeDtypeStruct((N, D), _f32),
      mesh=_mesh(),
      compiler_params=pltpu.CompilerParams(needs_layout_passes=False),
      scratch_types=[
          pltpu.VMEM((DCH,), jnp.int32),
          pltpu.VMEM((LCAP,), jnp.int32),
          pltpu.VMEM((LCAP,), jnp.int32),
          pltpu.VMEM((SCH, D), _f32),
          pltpu.VMEM((ACC_ROWS, D), _f32),
          pltpu.SemaphoreType.DMA,
      ],
  )


def _block(feat, ng128, src, dst, W1, b1, W2, b2, W3, b3):
  Wa = W1[0:D]
  Wb = W1[D:2 * D]
  Wcd = jnp.concatenate(
      [W1[2 * D:], jnp.zeros((GF - (W1.shape[0] - 2 * D), D), _f32)], axis=0)
  Ta, Tb = _node_mm()(feat, Wa, Wb, b1.reshape(1, D), ng128)
  Ga, Gb = _gather()(Ta, Tb, src, dst)
  m = _edge_mlp()(Ga, Gb, Wcd, W2, b2.reshape(1, D), W3, b3.reshape(1, D))
  return _scatter()(m, dst)


def kernel(x, pos, edge_index, lframes, batch,
           W1_0, b1_0, W2_0, b2_0, W3_0, b3_0,
           W1_1, b1_1, W2_1, b2_1, W3_1, b3_1):
  del batch
  src = edge_index[0]
  dst = edge_index[1]
  ng128 = jnp.concatenate(
      [pos, lframes.reshape(N, 9), jnp.zeros((N, 116), _f32)], axis=1)
  h = _block(x, ng128, src, dst, W1_0, b1_0, W2_0, b2_0, W3_0, b3_0)
  out = _block(h, ng128, src, dst, W1_1, b1_1, W2_1, b2_1, W3_1, b3_1)
  return out


# speedup vs baseline: 2.1019x; 1.1678x over previous
import functools

import jax
import jax.numpy as jnp
from jax import lax
from jax.experimental import pallas as pl
from jax.experimental.pallas import tpu as pltpu
from jax.experimental.pallas import tpu_sc as plsc

N = 10000
D = 256
E = 160000
NUM_RADIAL = 32
CUTOFF = 5.0
GW = 128
GF = 48
GEOW = 64

NC = 2
NS = 16
NW = NC * NS

EPW = E // NW
GCH = 40
GITER = EPW // GCH

NPW = 312
NPW_LAST = N - (NW - 1) * NPW
ACC_ROWS = 336
DUMMY = 332
DCH = 800
DITER = E // DCH
LCAP = 6400
SCH = 80

_f32 = jnp.float32
_bf16 = jnp.bfloat16


def _mesh():
  return plsc.VectorSubcoreMesh(
      core_axis_name="c", subcore_axis_name="s", num_cores=NC, num_subcores=NS)


def _pack2bf16(mat):
  u = lax.bitcast_convert_type(mat.astype(_bf16), jnp.uint16).astype(jnp.uint32)
  lo = u[:, :D // 2]
  hi = u[:, D // 2:]
  return lax.bitcast_convert_type(lo | (hi << 16), jnp.int32)


def _unpack2bf16(x):
  u = lax.bitcast_convert_type(x, jnp.uint32)
  lo = lax.bitcast_convert_type(u << 16, _f32)
  hi = lax.bitcast_convert_type(u & jnp.uint32(0xFFFF0000), _f32)
  return jnp.concatenate([lo, hi], axis=1)


def _node_mm_body(x_ref, wa_ref, wb_ref, b_ref, ta_ref, tb_ref):
  xv = x_ref[...]
  ta_ref[...] = _pack2bf16(jnp.dot(xv, wa_ref[...], preferred_element_type=_f32))
  tb_ref[...] = _pack2bf16(jnp.dot(xv, wb_ref[...], preferred_element_type=_f32)
                           + b_ref[...])


@functools.lru_cache(maxsize=None)
def _node_mm():
  TN = 1000
  grid = N // TN
  return pl.pallas_call(
      _node_mm_body,
      grid=(grid,),
      in_specs=[
          pl.BlockSpec((TN, D), lambda i: (i, 0)),
          pl.BlockSpec((D, D), lambda i: (0, 0)),
          pl.BlockSpec((D, D), lambda i: (0, 0)),
          pl.BlockSpec((1, D), lambda i: (0, 0)),
      ],
      out_specs=[
          pl.BlockSpec((TN, D // 2), lambda i: (i, 0)),
          pl.BlockSpec((TN, D // 2), lambda i: (i, 0)),
      ],
      out_shape=[
          jax.ShapeDtypeStruct((N, D // 2), jnp.int32),
          jax.ShapeDtypeStruct((N, D // 2), jnp.int32),
      ],
  )


def _gather_body(with_geo, *refs):
  if with_geo:
    (ta, tb, ng, src, dst, ga, gb, gs, gd,
     sidx, didx, bufa, bufb, bufs, bufd, sem) = refs
  else:
    (ta, tb, src, dst, ga, gb, sidx, didx, bufa, bufb, sem) = refs
  wid = lax.axis_index("s") * NC + lax.axis_index("c")
  base = wid * EPW

  def chunk(i, carry):
    off = base + i * GCH
    pltpu.sync_copy(src.at[pl.ds(off, GCH)], sidx)
    pltpu.sync_copy(dst.at[pl.ds(off, GCH)], didx)
    ca = pltpu.async_copy(ta.at[sidx], bufa, sem)
    cb = pltpu.async_copy(tb.at[didx], bufb, sem)
    if with_geo:
      cs = pltpu.async_copy(ng.at[sidx], bufs, sem)
      cd = pltpu.async_copy(ng.at[didx], bufd, sem)
    ca.wait()
    cb.wait()
    if with_geo:
      cs.wait()
      cd.wait()
    pltpu.sync_copy(bufa, ga.at[pl.ds(off, GCH)])
    pltpu.sync_copy(bufb, gb.at[pl.ds(off, GCH)])
    if with_geo:
      pltpu.sync_copy(bufs, gs.at[pl.ds(off, GCH)])
      pltpu.sync_copy(bufd, gd.at[pl.ds(off, GCH)])
    return carry

  lax.fori_loop(0, GITER, chunk, 0)


@functools.lru_cache(maxsize=None)
def _gather(with_geo):
  out_type = [
      jax.ShapeDtypeStruct((E, D // 2), jnp.int32),
      jax.ShapeDtypeStruct((E, D // 2), jnp.int32),
  ]
  scratch = [
      pltpu.VMEM((GCH,), jnp.int32),
      pltpu.VMEM((GCH,), jnp.int32),
      pltpu.VMEM((GCH, D // 2), jnp.int32),
      pltpu.VMEM((GCH, D // 2), jnp.int32),
  ]
  if with_geo:
    out_type += [
        jax.ShapeDtypeStruct((E, GW), _f32),
        jax.ShapeDtypeStruct((E, GW), _f32),
    ]
    scratch += [
        pltpu.VMEM((GCH, GW), _f32),
        pltpu.VMEM((GCH, GW), _f32),
    ]
  scratch.append(pltpu.SemaphoreType.DMA)
  return pl.kernel(
      functools.partial(_gather_body, with_geo),
      out_type=out_type,
      mesh=_mesh(),
      compiler_params=pltpu.CompilerParams(needs_layout_passes=False),
      scratch_types=scratch,
  )


def _silu(v):
  return v * (1.0 / (1.0 + jnp.exp(-v)))


def _geoT_from(gs, gd):
  gsT = gs[:, 0:16].T
  gdT = gd[:, 0:16].T
  vec = gsT[0:3, :] - gdT[0:3, :]
  L = gdT[3:12, :]
  v0 = vec[0:1, :]
  v1 = vec[1:2, :]
  v2 = vec[2:3, :]
  vl0 = L[0:1, :] * v0 + L[1:2, :] * v1 + L[2:3, :] * v2
  vl1 = L[3:4, :] * v0 + L[4:5, :] * v1 + L[5:6, :] * v2
  vl2 = L[6:7, :] * v0 + L[7:8, :] * v1 + L[8:9, :] * v2
  r2 = vl0 * vl0 + vl1 * vl1 + vl2 * vl2
  r = jnp.sqrt(r2)
  rinv = 1.0 / (r + 1e-8)
  u0 = vl0 * rinv
  u1 = vl1 * rinv
  u2 = vl2 * rinv
  centers = lax.broadcasted_iota(jnp.int32, (NUM_RADIAL, 1), 0).astype(_f32) * (
      CUTOFF / (NUM_RADIAL - 1))
  dr = r - centers
  rad = jnp.exp(-10.0 * dr * dr)
  us = [u0, u1, u2]
  outer = [us[i] * us[j] for i in range(3) for j in range(3)]
  te = r.shape[1]
  return jnp.concatenate(
      [rad, u0, u1, u2] + outer + [jnp.zeros((4, te), _f32)], axis=0)


def _mlp_tail(geoT, ga_ref, gb_ref, w1_ref, w2_ref, b2_ref, w3_ref, b3_ref,
              m_ref):
  term = lax.dot_general(geoT, w1_ref[...], (((0,), (0,)), ((), ())),
                         preferred_element_type=_f32)
  pre1 = _unpack2bf16(ga_ref[...]) + _unpack2bf16(gb_ref[...]) + term
  h1 = _silu(pre1)
  h2 = _silu(jnp.dot(h1, w2_ref[...], preferred_element_type=_f32) + b2_ref[...])
  m_ref[...] = jnp.dot(h2, w3_ref[...], preferred_element_type=_f32) + b3_ref[...]


def _edge_mlp0_body(ga_ref, gb_ref, gs_ref, gd_ref, w1_ref, w2_ref, b2_ref,
                    w3_ref, b3_ref, m_ref, geo_ref):
  geoT = _geoT_from(gs_ref[...], gd_ref[...])
  geo_ref[0:GF, :] = geoT
  geo_ref[GF:, :] = jnp.zeros((GEOW - GF, geoT.shape[1]), _f32)
  _mlp_tail(geoT, ga_ref, gb_ref, w1_ref, w2_ref, b2_ref, w3_ref, b3_ref, m_ref)


def _edge_mlp1_body(ga_ref, gb_ref, geo_ref, w1_ref, w2_ref, b2_ref,
                    w3_ref, b3_ref, m_ref):
  geoT = geo_ref[0:GF, :]
  _mlp_tail(geoT, ga_ref, gb_ref, w1_ref, w2_ref, b2_ref, w3_ref, b3_ref, m_ref)


@functools.lru_cache(maxsize=None)
def _edge_mlp(first):
  TE = 3200
  grid = E // TE
  wspecs = [
      pl.BlockSpec((GF, D), lambda i: (0, 0)),
      pl.BlockSpec((D, D), lambda i: (0, 0)),
      pl.BlockSpec((1, D), lambda i: (0, 0)),
      pl.BlockSpec((D, D), lambda i: (0, 0)),
      pl.BlockSpec((1, D), lambda i: (0, 0)),
  ]
  gspecs = [
      pl.BlockSpec((TE, D // 2), lambda i: (i, 0)),
      pl.BlockSpec((TE, D // 2), lambda i: (i, 0)),
  ]
  if first:
    return pl.pallas_call(
        _edge_mlp0_body,
        grid=(grid,),
        in_specs=gspecs + [
            pl.BlockSpec((TE, GW), lambda i: (i, 0)),
            pl.BlockSpec((TE, GW), lambda i: (i, 0)),
        ] + wspecs,
        out_specs=[
            pl.BlockSpec((TE, D), lambda i: (i, 0)),
            pl.BlockSpec((GEOW, TE), lambda i: (0, i)),
        ],
        out_shape=[
            jax.ShapeDtypeStruct((E, D), _f32),
            jax.ShapeDtypeStruct((GEOW, E), _f32),
        ],
    )
  return pl.pallas_call(
      _edge_mlp1_body,
      grid=(grid,),
      in_specs=gspecs + [
          pl.BlockSpec((GEOW, TE), lambda i: (0, i)),
      ] + wspecs,
      out_specs=pl.BlockSpec((TE, D), lambda i: (i, 0)),
      out_shape=jax.ShapeDtypeStruct((E, D), _f32),
  )


def _scatter_body(m, dst, out, didx, eids, lids, gbuf, acc, sem):
  w = lax.axis_index("s") * NC + lax.axis_index("c")
  nodebase = w * NPW

  zero16 = jnp.zeros((16,), _f32)

  def zrow(r, carry):
    for jj in range(D // 16):
      acc[r, pl.ds(jj * 16, 16)] = zero16
    return carry

  lax.fori_loop(0, ACC_ROWS, zrow, 0)

  iota16 = lax.iota(jnp.int32, 16)

  bound = jnp.where(w == NW - 1, NPW_LAST, NPW)

  def scan_chunk(i, off):
    cbase = i * DCH
    pltpu.sync_copy(dst.at[pl.ds(cbase, DCH)], didx)

    def group(g, off):
      v = didx[pl.ds(g * 16, 16)] - nodebase
      ok = (v >= 0) & (v < bound)
      okx = ok.astype(jnp.int32)
      csum = plsc.cumsum(okx)
      pos = jnp.where(ok, off + csum - okx, LCAP - 1)
      plsc.store_scatter(eids, [pos], cbase + g * 16 + iota16)
      plsc.store_scatter(lids, [pos], v)
      return off + jnp.sum(okx)

    return lax.fori_loop(0, DCH // 16, group, off)

  cnt = lax.fori_loop(0, DITER, scan_chunk, 0)

  for g in range(SCH // 16):
    eids[pl.ds(cnt + g * 16, 16)] = jnp.zeros((16,), jnp.int32)
    lids[pl.ds(cnt + g * 16, 16)] = jnp.full((16,), DUMMY, jnp.int32)

  nchunk = (cnt + SCH - 1) // SCH

  def acc_chunk(ch, carry):
    pltpu.async_copy(m.at[eids.at[pl.ds(ch * SCH, SCH)]], gbuf, sem).wait()

    def row(k, carry):
      d = lids[pl.ds(ch * SCH + k, 16)][0]
      for jj in range(D // 16):
        plsc.addupdate(acc.at[d, pl.ds(jj * 16, 16)],
                       gbuf[k, pl.ds(jj * 16, 16)])
      return carry

    lax.fori_loop(0, SCH, row, 0)
    return carry

  lax.fori_loop(0, nchunk, acc_chunk, 0)

  @pl.when(w < NW - 1)
  def _():
    pltpu.sync_copy(acc.at[pl.ds(0, NPW)], out.at[pl.ds(nodebase, NPW)])

  @pl.when(w == NW - 1)
  def _():
    pltpu.sync_copy(acc.at[pl.ds(0, NPW_LAST)],
                    out.at[pl.ds(nodebase, NPW_LAST)])


@functools.lru_cache(maxsize=None)
def _scatter():
  return pl.kernel(
      _scatter_body,
      out_type=jax.ShapeDtypeStruct((N, D), _f32),
      mesh=_mesh(),
      compiler_params=pltpu.CompilerParams(needs_layout_passes=False),
      scratch_types=[
          pltpu.VMEM((DCH,), jnp.int32),
          pltpu.VMEM((LCAP,), jnp.int32),
          pltpu.VMEM((LCAP,), jnp.int32),
          pltpu.VMEM((SCH, D), _f32),
          pltpu.VMEM((ACC_ROWS, D), _f32),
          pltpu.SemaphoreType.DMA,
      ],
  )


def _block(feat, ng128, src, dst, geo, W1, b1, W2, b2, W3, b3):
  Wa = W1[0:D]
  Wb = W1[D:2 * D]
  Wcd = jnp.concatenate(
      [W1[2 * D:], jnp.zeros((GF - (W1.shape[0] - 2 * D), D), _f32)], axis=0)
  Ta, Tb = _node_mm()(feat, Wa, Wb, b1.reshape(1, D))
  if geo is None:
    Ga, Gb, Gs, Gd = _gather(True)(Ta, Tb, ng128, src, dst)
    m, geo = _edge_mlp(True)(Ga, Gb, Gs, Gd, Wcd, W2, b2.reshape(1, D), W3,
                             b3.reshape(1, D))
  else:
    Ga, Gb = _gather(False)(Ta, Tb, src, dst)
    m = _edge_mlp(False)(Ga, Gb, geo, Wcd, W2, b2.reshape(1, D), W3,
                         b3.reshape(1, D))
  return _scatter()(m, dst), geo


def kernel(x, pos, edge_index, lframes, batch,
           W1_0, b1_0, W2_0, b2_0, W3_0, b3_0,
           W1_1, b1_1, W2_1, b2_1, W3_1, b3_1):
  del batch
  src = edge_index[0]
  dst = edge_index[1]
  ng128 = jnp.concatenate(
      [pos, lframes.reshape(N, 9), jnp.zeros((N, GW - 12), _f32)], axis=1)
  h, geo = _block(x, ng128, src, dst, None,
                  W1_0, b1_0, W2_0, b2_0, W3_0, b3_0)
  out, _ = _block(h, ng128, src, dst, geo,
                  W1_1, b1_1, W2_1, b2_1, W3_1, b3_1)
  return out


# speedup vs baseline: 2.5561x; 1.2161x over previous
import functools

import jax
import jax.numpy as jnp
from jax import lax
from jax.experimental import pallas as pl
from jax.experimental.pallas import tpu as pltpu
from jax.experimental.pallas import tpu_sc as plsc

N = 10000
D = 256
E = 160000
NUM_RADIAL = 32
CUTOFF = 5.0
GW = 128
GF = 48
GEOW = 64

NC = 2
NS = 16
NW = NC * NS

EPW = E // NW
GCH = 40
GITER = EPW // GCH

NPW = 312
NPW_LAST = N - (NW - 1) * NPW
ACC_ROWS = 336
DUMMY = 332
DCH = 800
DITER = E // DCH
LCAP = 5696
SCH = 48

_f32 = jnp.float32
_bf16 = jnp.bfloat16


def _mesh():
  return plsc.VectorSubcoreMesh(
      core_axis_name="c", subcore_axis_name="s", num_cores=NC, num_subcores=NS)


def _pack2bf16(mat):
  u = lax.bitcast_convert_type(mat.astype(_bf16), jnp.uint16).astype(jnp.uint32)
  lo = u[:, :D // 2]
  hi = u[:, D // 2:]
  return lax.bitcast_convert_type(lo | (hi << 16), jnp.int32)


def _unpack2bf16(x):
  u = lax.bitcast_convert_type(x, jnp.uint32)
  lo = lax.bitcast_convert_type(u << 16, _f32)
  hi = lax.bitcast_convert_type(u & jnp.uint32(0xFFFF0000), _f32)
  return jnp.concatenate([lo, hi], axis=1)


def _node_mm_body(x_ref, wa_ref, wb_ref, b_ref, ta_ref, tb_ref):
  xv = x_ref[...]
  ta_ref[...] = _pack2bf16(jnp.dot(xv, wa_ref[...], preferred_element_type=_f32))
  tb_ref[...] = _pack2bf16(jnp.dot(xv, wb_ref[...], preferred_element_type=_f32)
                           + b_ref[...])


@functools.lru_cache(maxsize=None)
def _node_mm():
  TN = 1000
  grid = N // TN
  return pl.pallas_call(
      _node_mm_body,
      grid=(grid,),
      in_specs=[
          pl.BlockSpec((TN, D), lambda i: (i, 0)),
          pl.BlockSpec((D, D), lambda i: (0, 0)),
          pl.BlockSpec((D, D), lambda i: (0, 0)),
          pl.BlockSpec((1, D), lambda i: (0, 0)),
      ],
      out_specs=[
          pl.BlockSpec((TN, D // 2), lambda i: (i, 0)),
          pl.BlockSpec((TN, D // 2), lambda i: (i, 0)),
      ],
      out_shape=[
          jax.ShapeDtypeStruct((N, D // 2), jnp.int32),
          jax.ShapeDtypeStruct((N, D // 2), jnp.int32),
      ],
  )


def _gather_body(with_geo, *refs):
  if with_geo:
    (ta, tb, ng, src, dst, ga, gb, gs, gd,
     sidx, didx, bufa, bufb, bufs, bufd, sem) = refs
  else:
    (ta, tb, src, dst, ga, gb, sidx, didx, bufa, bufb, sem) = refs
  wid = lax.axis_index("s") * NC + lax.axis_index("c")
  base = wid * EPW

  def chunk(i, carry):
    off = base + i * GCH
    pltpu.sync_copy(src.at[pl.ds(off, GCH)], sidx)
    pltpu.sync_copy(dst.at[pl.ds(off, GCH)], didx)
    ca = pltpu.async_copy(ta.at[sidx], bufa, sem)
    cb = pltpu.async_copy(tb.at[didx], bufb, sem)
    if with_geo:
      cs = pltpu.async_copy(ng.at[sidx], bufs, sem)
      cd = pltpu.async_copy(ng.at[didx], bufd, sem)
    ca.wait()
    cb.wait()
    if with_geo:
      cs.wait()
      cd.wait()
    pltpu.sync_copy(bufa, ga.at[pl.ds(off, GCH)])
    pltpu.sync_copy(bufb, gb.at[pl.ds(off, GCH)])
    if with_geo:
      pltpu.sync_copy(bufs, gs.at[pl.ds(off, GCH)])
      pltpu.sync_copy(bufd, gd.at[pl.ds(off, GCH)])
    return carry

  lax.fori_loop(0, GITER, chunk, 0)


@functools.lru_cache(maxsize=None)
def _gather(with_geo):
  out_type = [
      jax.ShapeDtypeStruct((E, D // 2), jnp.int32),
      jax.ShapeDtypeStruct((E, D // 2), jnp.int32),
  ]
  scratch = [
      pltpu.VMEM((GCH,), jnp.int32),
      pltpu.VMEM((GCH,), jnp.int32),
      pltpu.VMEM((GCH, D // 2), jnp.int32),
      pltpu.VMEM((GCH, D // 2), jnp.int32),
  ]
  if with_geo:
    out_type += [
        jax.ShapeDtypeStruct((E, GW), _f32),
        jax.ShapeDtypeStruct((E, GW), _f32),
    ]
    scratch += [
        pltpu.VMEM((GCH, GW), _f32),
        pltpu.VMEM((GCH, GW), _f32),
    ]
  scratch.append(pltpu.SemaphoreType.DMA)
  return pl.kernel(
      functools.partial(_gather_body, with_geo),
      out_type=out_type,
      mesh=_mesh(),
      compiler_params=pltpu.CompilerParams(needs_layout_passes=False),
      scratch_types=scratch,
  )


def _silu(v):
  return v * (1.0 / (1.0 + jnp.exp(-v)))


def _geoT_from(gs, gd):
  gsT = gs[:, 0:16].T
  gdT = gd[:, 0:16].T
  vec = gsT[0:3, :] - gdT[0:3, :]
  L = gdT[3:12, :]
  v0 = vec[0:1, :]
  v1 = vec[1:2, :]
  v2 = vec[2:3, :]
  vl0 = L[0:1, :] * v0 + L[1:2, :] * v1 + L[2:3, :] * v2
  vl1 = L[3:4, :] * v0 + L[4:5, :] * v1 + L[5:6, :] * v2
  vl2 = L[6:7, :] * v0 + L[7:8, :] * v1 + L[8:9, :] * v2
  r2 = vl0 * vl0 + vl1 * vl1 + vl2 * vl2
  r = jnp.sqrt(r2)
  rinv = 1.0 / (r + 1e-8)
  u0 = vl0 * rinv
  u1 = vl1 * rinv
  u2 = vl2 * rinv
  centers = lax.broadcasted_iota(jnp.int32, (NUM_RADIAL, 1), 0).astype(_f32) * (
      CUTOFF / (NUM_RADIAL - 1))
  dr = r - centers
  rad = jnp.exp(-10.0 * dr * dr)
  us = [u0, u1, u2]
  outer = [us[i] * us[j] for i in range(3) for j in range(3)]
  te = r.shape[1]
  return jnp.concatenate(
      [rad, u0, u1, u2] + outer + [jnp.zeros((4, te), _f32)], axis=0)


def _mlp_tail(geoT, ga_ref, gb_ref, w1_ref, w2_ref, b2_ref, w3_ref, b3_ref,
              m_ref):
  term = lax.dot_general(geoT, w1_ref[...], (((0,), (0,)), ((), ())),
                         preferred_element_type=_f32)
  pre1 = _unpack2bf16(ga_ref[...]) + _unpack2bf16(gb_ref[...]) + term
  h1 = _silu(pre1)
  h2 = _silu(jnp.dot(h1, w2_ref[...], preferred_element_type=_f32) + b2_ref[...])
  m_ref[...] = jnp.dot(h2, w3_ref[...], preferred_element_type=_f32) + b3_ref[...]


def _edge_mlp0_body(ga_ref, gb_ref, gs_ref, gd_ref, w1_ref, w2_ref, b2_ref,
                    w3_ref, b3_ref, m_ref, geo_ref):
  geoT = _geoT_from(gs_ref[...], gd_ref[...])
  geo_ref[0:GF, :] = geoT
  geo_ref[GF:, :] = jnp.zeros((GEOW - GF, geoT.shape[1]), _f32)
  _mlp_tail(geoT, ga_ref, gb_ref, w1_ref, w2_ref, b2_ref, w3_ref, b3_ref, m_ref)


def _edge_mlp1_body(ga_ref, gb_ref, geo_ref, w1_ref, w2_ref, b2_ref,
                    w3_ref, b3_ref, m_ref):
  geoT = geo_ref[0:GF, :]
  _mlp_tail(geoT, ga_ref, gb_ref, w1_ref, w2_ref, b2_ref, w3_ref, b3_ref, m_ref)


@functools.lru_cache(maxsize=None)
def _edge_mlp(first):
  TE = 3200
  grid = E // TE
  wspecs = [
      pl.BlockSpec((GF, D), lambda i: (0, 0)),
      pl.BlockSpec((D, D), lambda i: (0, 0)),
      pl.BlockSpec((1, D), lambda i: (0, 0)),
      pl.BlockSpec((D, D), lambda i: (0, 0)),
      pl.BlockSpec((1, D), lambda i: (0, 0)),
  ]
  gspecs = [
      pl.BlockSpec((TE, D // 2), lambda i: (i, 0)),
      pl.BlockSpec((TE, D // 2), lambda i: (i, 0)),
  ]
  if first:
    return pl.pallas_call(
        _edge_mlp0_body,
        grid=(grid,),
        in_specs=gspecs + [
            pl.BlockSpec((TE, GW), lambda i: (i, 0)),
            pl.BlockSpec((TE, GW), lambda i: (i, 0)),
        ] + wspecs,
        out_specs=[
            pl.BlockSpec((TE, D), lambda i: (i, 0)),
            pl.BlockSpec((GEOW, TE), lambda i: (0, i)),
        ],
        out_shape=[
            jax.ShapeDtypeStruct((E, D), _f32),
            jax.ShapeDtypeStruct((GEOW, E), _f32),
        ],
    )
  return pl.pallas_call(
      _edge_mlp1_body,
      grid=(grid,),
      in_specs=gspecs + [
          pl.BlockSpec((GEOW, TE), lambda i: (0, i)),
      ] + wspecs,
      out_specs=pl.BlockSpec((TE, D), lambda i: (i, 0)),
      out_shape=jax.ShapeDtypeStruct((E, D), _f32),
  )


def _scan16(a, iota16):
  dnums = lax.GatherDimensionNumbers(offset_dims=(), collapsed_slice_dims=(0,),
                                     start_index_map=(0,))
  p = a
  for k in (1, 2, 4, 8):
    sh = lax.gather(p, ((iota16 - k) % 16).reshape(16, 1), dnums, (1,),
                    mode=lax.GatherScatterMode.PROMISE_IN_BOUNDS)
    p = p + jnp.where(iota16 >= k, sh, 0)
  return p


def _scatter_body(build, *refs):
  if build:
    (m, dst, out, eids_hbm, lids_hbm, cnts_hbm,
     didxb, eids, lids, gbuf, cntv, acc, semd, semg) = refs
  else:
    (m, eids_hbm, lids_hbm, cnts_hbm, out,
     didxb, eids, lids, gbuf, cntv, acc, semd, semg) = refs
  w = lax.axis_index("s") * NC + lax.axis_index("c")
  nodebase = w * NPW

  zero16 = jnp.zeros((16,), _f32)

  def zrow(r, carry):
    for jj in range(D // 16):
      acc[r, pl.ds(jj * 16, 16)] = zero16
    return carry

  lax.fori_loop(0, ACC_ROWS, zrow, 0)

  iota16 = lax.iota(jnp.int32, 16)

  if build:
    bound = jnp.where(w == NW - 1, NPW_LAST, NPW)

    pltpu.async_copy(dst.at[pl.ds(0, DCH)], didxb.at[pl.ds(0, DCH)], semd)

    def scan_chunk(i, off):
      b = i % 2

      @pl.when(i + 1 < DITER)
      def _():
        pltpu.async_copy(dst.at[pl.ds((i + 1) * DCH, DCH)],
                         didxb.at[pl.ds((1 - b) * DCH, DCH)], semd)

      pltpu.make_async_copy(dst.at[pl.ds(i * DCH, DCH)],
                            didxb.at[pl.ds(b * DCH, DCH)], semd).wait()
      cbase = i * DCH

      def group(g, off):
        v = didxb[pl.ds(b * DCH + g * 16, 16)] - nodebase
        ok = (v >= 0) & (v < bound)
        a = ok.astype(jnp.int32)
        p = _scan16(a, iota16)
        pos = jnp.where(ok, off + p - a, LCAP - 1)
        plsc.store_scatter(eids, [pos], cbase + g * 16 + iota16)
        plsc.store_scatter(lids, [pos], v)
        return off + p[15]

      return lax.fori_loop(0, DCH // 16, group, off)

    cnt = lax.fori_loop(0, DITER, scan_chunk, 0)

    for g in range(SCH // 16):
      eids[pl.ds(cnt + g * 16, 16)] = jnp.zeros((16,), jnp.int32)
      lids[pl.ds(cnt + g * 16, 16)] = jnp.full((16,), DUMMY, jnp.int32)
    cntv[pl.ds(0, 16)] = jnp.full((16,), cnt, jnp.int32)
    pltpu.sync_copy(eids, eids_hbm.at[pl.ds(w * LCAP, LCAP)])
    pltpu.sync_copy(lids, lids_hbm.at[pl.ds(w * LCAP, LCAP)])
    pltpu.sync_copy(cntv, cnts_hbm.at[pl.ds(w * 16, 16)])
  else:
    pltpu.sync_copy(eids_hbm.at[pl.ds(w * LCAP, LCAP)], eids)
    pltpu.sync_copy(lids_hbm.at[pl.ds(w * LCAP, LCAP)], lids)
    pltpu.sync_copy(cnts_hbm.at[pl.ds(w * 16, 16)], cntv)
    cnt = cntv[pl.ds(0, 16)][0]

  nchunk = (cnt + SCH - 1) // SCH

  @pl.when(nchunk > 0)
  def _():
    pltpu.async_copy(m.at[eids.at[pl.ds(0, SCH)]], gbuf.at[pl.ds(0, SCH)], semg)

  def acc_chunk(ch, carry):
    b = ch % 2

    @pl.when(ch + 1 < nchunk)
    def _():
      pltpu.async_copy(m.at[eids.at[pl.ds((ch + 1) * SCH, SCH)]],
                       gbuf.at[pl.ds((1 - b) * SCH, SCH)], semg)

    pltpu.make_async_copy(m.at[eids.at[pl.ds(ch * SCH, SCH)]],
                          gbuf.at[pl.ds(b * SCH, SCH)], semg).wait()

    def row(k, carry):
      d = lids[pl.ds(ch * SCH + k, 16)][0]
      for jj in range(D // 16):
        plsc.addupdate(acc.at[d, pl.ds(jj * 16, 16)],
                       gbuf[b * SCH + k, pl.ds(jj * 16, 16)])
      return carry

    lax.fori_loop(0, SCH, row, 0)
    return carry

  lax.fori_loop(0, nchunk, acc_chunk, 0)

  @pl.when(w < NW - 1)
  def _():
    pltpu.sync_copy(acc.at[pl.ds(0, NPW)], out.at[pl.ds(nodebase, NPW)])

  @pl.when(w == NW - 1)
  def _():
    pltpu.sync_copy(acc.at[pl.ds(0, NPW_LAST)],
                    out.at[pl.ds(nodebase, NPW_LAST)])


@functools.lru_cache(maxsize=None)
def _scatter(build):
  if build:
    out_type = [
        jax.ShapeDtypeStruct((N, D), _f32),
        jax.ShapeDtypeStruct((NW * LCAP,), jnp.int32),
        jax.ShapeDtypeStruct((NW * LCAP,), jnp.int32),
        jax.ShapeDtypeStruct((NW * 16,), jnp.int32),
    ]
  else:
    out_type = jax.ShapeDtypeStruct((N, D), _f32)
  return pl.kernel(
      functools.partial(_scatter_body, build),
      out_type=out_type,
      mesh=_mesh(),
      compiler_params=pltpu.CompilerParams(needs_layout_passes=False),
      scratch_types=[
          pltpu.VMEM((2 * DCH,), jnp.int32),
          pltpu.VMEM((LCAP,), jnp.int32),
          pltpu.VMEM((LCAP,), jnp.int32),
          pltpu.VMEM((2 * SCH, D), _f32),
          pltpu.VMEM((16,), jnp.int32),
          pltpu.VMEM((ACC_ROWS, D), _f32),
          pltpu.SemaphoreType.DMA,
          pltpu.SemaphoreType.DMA,
      ],
  )


def _block(feat, ng128, src, dst, geo, lists, W1, b1, W2, b2, W3, b3):
  Wa = W1[0:D]
  Wb = W1[D:2 * D]
  Wcd = jnp.concatenate(
      [W1[2 * D:], jnp.zeros((GF - (W1.shape[0] - 2 * D), D), _f32)], axis=0)
  Ta, Tb = _node_mm()(feat, Wa, Wb, b1.reshape(1, D))
  if geo is None:
    Ga, Gb, Gs, Gd = _gather(True)(Ta, Tb, ng128, src, dst)
    m, geo = _edge_mlp(True)(Ga, Gb, Gs, Gd, Wcd, W2, b2.reshape(1, D), W3,
                             b3.reshape(1, D))
    out, el, ll, cl = _scatter(True)(m, dst)
    return out, geo, (el, ll, cl)
  Ga, Gb = _gather(False)(Ta, Tb, src, dst)
  m = _edge_mlp(False)(Ga, Gb, geo, Wcd, W2, b2.reshape(1, D), W3,
                       b3.reshape(1, D))
  el, ll, cl = lists
  return _scatter(False)(m, el, ll, cl), geo, lists


def kernel(x, pos, edge_index, lframes, batch,
           W1_0, b1_0, W2_0, b2_0, W3_0, b3_0,
           W1_1, b1_1, W2_1, b2_1, W3_1, b3_1):
  del batch
  src = edge_index[0]
  dst = edge_index[1]
  ng128 = jnp.concatenate(
      [pos, lframes.reshape(N, 9), jnp.zeros((N, GW - 12), _f32)], axis=1)
  h, geo, lists = _block(x, ng128, src, dst, None, None,
                         W1_0, b1_0, W2_0, b2_0, W3_0, b3_0)
  out, _, _ = _block(h, ng128, src, dst, geo, lists,
                     W1_1, b1_1, W2_1, b2_1, W3_1, b3_1)
  return out
